# baseline (speedup 1.0000x reference)
import functools

import jax
import jax.numpy as jnp
from jax import lax
from jax.experimental import pallas as pl
from jax.experimental.pallas import tpu as pltpu
from jax.experimental.pallas import tpu_sc as plsc

B, T = 16384, 200
NC, NS, L = 2, 16, 16
NW = NC * NS
RPW = B // NW
CHUNK = 128
NCHUNK = RPW // CHUNK
GROUPS = CHUNK // L

_OUT = tuple(jax.ShapeDtypeStruct((B, T), jnp.float32) for _ in range(5))


@functools.partial(
    pl.kernel,
    out_type=_OUT,
    mesh=plsc.VectorSubcoreMesh(core_axis_name="c", subcore_axis_name="s"),
    scratch_types=[
        pltpu.VMEM((CHUNK, T), jnp.float32),
        pltpu.VMEM((CHUNK, T), jnp.float32),
        pltpu.SemaphoreType.DMA,
        pltpu.SemaphoreType.DMA,
        pltpu.SemaphoreType.DMA,
    ],
    compiler_params=pltpu.CompilerParams(
        use_tc_tiling_on_sc=True, needs_layout_passes=False
    ),
)
def _perturb(x_hbm, o0, o1, o2, o3, o4, bufA, bufB, sem_in, sem_sw, sem_st):
    wid = lax.axis_index("s") * NC + lax.axis_index("c")
    lanes = lax.iota(jnp.int32, L)
    col0 = jnp.zeros((L,), jnp.int32)
    col1 = col0 + 1

    def swap_inplace(buf):
        for g in range(GROUPS):
            rows = lanes + (g * L)
            v0 = plsc.load_gather(buf, [rows, col0])
            v1 = plsc.load_gather(buf, [rows, col1])
            plsc.store_scatter(buf, [rows, col0], v1)
            plsc.store_scatter(buf, [rows, col1], v0)

    bufs = [bufA, bufB]
    rows = [pl.ds(wid * RPW + i * CHUNK, CHUNK) for i in range(NCHUNK)]
    ins = [None] * NCHUNK
    straight = [None] * NCHUNK
    ins[0] = pltpu.async_copy(x_hbm.at[rows[0]], bufs[0], sem_in)
    for i in range(NCHUNK):
        X = bufs[i % 2]
        ins[i].wait()
        if i >= 1:
            for h in straight[i - 1]:
                h.wait()
        if i + 1 < NCHUNK:
            ins[i + 1] = pltpu.async_copy(
                x_hbm.at[rows[i + 1]], bufs[(i + 1) % 2], sem_in
            )
        swap_inplace(X)
        w1 = pltpu.async_copy(X, o1.at[rows[i]], sem_sw)
        w3 = pltpu.async_copy(X, o3.at[rows[i]], sem_sw)
        w1.wait()
        w3.wait()
        swap_inplace(X)
        straight[i] = [
            pltpu.async_copy(X, o.at[rows[i]], sem_st) for o in (o0, o2, o4)
        ]
    for h in straight[NCHUNK - 1]:
        h.wait()


def kernel(x):
    return _perturb(x)


# speedup vs baseline: 1.3444x; 1.3444x over previous
import functools

import jax
import jax.numpy as jnp
from jax import lax
from jax.experimental import pallas as pl
from jax.experimental.pallas import tpu as pltpu
from jax.experimental.pallas import tpu_sc as plsc

B, T = 16384, 200
NC, NS, L = 2, 16, 16
NW = NC * NS
RPW = B // NW
CHUNK = 256
NCHUNK = RPW // CHUNK
GROUPS = CHUNK // L


@functools.partial(
    pl.kernel,
    out_type=jax.ShapeDtypeStruct((B, T), jnp.float32),
    mesh=plsc.VectorSubcoreMesh(core_axis_name="c", subcore_axis_name="s"),
    scratch_types=[pltpu.VMEM((CHUNK, T), jnp.float32)],
    compiler_params=pltpu.CompilerParams(
        use_tc_tiling_on_sc=True, needs_layout_passes=False
    ),
)
def _swap01(x_hbm, y_hbm, buf):
    wid = lax.axis_index("s") * NC + lax.axis_index("c")
    lanes = lax.iota(jnp.int32, L)
    col0 = jnp.zeros((L,), jnp.int32)
    col1 = col0 + 1
    for ch in range(NCHUNK):
        base = wid * RPW + ch * CHUNK
        pltpu.sync_copy(x_hbm.at[pl.ds(base, CHUNK)], buf)
        for g in range(GROUPS):
            rows = lanes + (g * L)
            v0 = plsc.load_gather(buf, [rows, col0])
            v1 = plsc.load_gather(buf, [rows, col1])
            plsc.store_scatter(buf, [rows, col0], v1)
            plsc.store_scatter(buf, [rows, col1], v0)
        pltpu.sync_copy(buf, y_hbm.at[pl.ds(base, CHUNK)])


def kernel(x):
    y = _swap01(x)
    return (x, y, x, y, x)
